# Initial kernel scaffold; baseline (speedup 1.0000x reference)
import jax
import jax.numpy as jnp
from jax.experimental import pallas as pl


def kernel(user_dense_input, user_sparse_input, item_dense_input, item_sparse_input, behavior_input, emb_user_id, emb_user_city, emb_user_age, emb_item, emb_cate, att_W0, att_b0, att_a0, att_W1, att_b1, att_a1, att_Wf, att_bf, bn_gamma, bn_beta, ffn_W0, ffn_b0, ffn_a0, ffn_W1, ffn_b1, ffn_a1, out_W, out_b):
    raise NotImplementedError("write your pallas kernel here")


# speedup vs baseline: 1.7442x; 1.7442x over previous
import functools
import math

import jax
import jax.numpy as jnp
from jax import lax
from jax.experimental import pallas as pl
from jax.experimental.pallas import tpu as pltpu
from jax.experimental.pallas import tpu_sc as plsc

T = 20
NW = 32
CH = 128


def _sc_gather_all(u0, u1, u2, qit, qct, bit, bct,
                   emb_user_id, emb_user_city, emb_user_age,
                   emb_item, emb_cate):
  B = u0.shape[0]
  nq = (B // NW) // CH
  nb = (T * B // NW) // CH
  mesh = plsc.VectorSubcoreMesh(core_axis_name="c", subcore_axis_name="s")

  out_type = [
      jax.ShapeDtypeStruct((B, 32), jnp.float32),
      jax.ShapeDtypeStruct((B, 32), jnp.float32),
      jax.ShapeDtypeStruct((B, 32), jnp.float32),
      jax.ShapeDtypeStruct((B, 64), jnp.float32),
      jax.ShapeDtypeStruct((B, 64), jnp.float32),
      jax.ShapeDtypeStruct((T * B, 64), jnp.float32),
      jax.ShapeDtypeStruct((T * B, 64), jnp.float32),
  ]
  scratch_types = [
      pltpu.VMEM((CH,), jnp.int32),
      pltpu.VMEM((CH, 32), jnp.float32),
      pltpu.VMEM((CH, 64), jnp.float32),
      pltpu.SemaphoreType.DMA,
  ]

  @functools.partial(pl.kernel, out_type=out_type, mesh=mesh,
                     scratch_types=scratch_types,
                     compiler_params=pltpu.CompilerParams(
                         use_tc_tiling_on_sc=False))
  def k(u0h, u1h, u2h, qih, qch, bih, bch,
        t_u0, t_u1, t_u2, t_it, t_ct,
        ue0o, ue1o, ue2o, qio, qco, bio, bco,
        idx_v, rows32, rows64, sem):
    wid = lax.axis_index("s") * 2 + lax.axis_index("c")

    def gather_pass(idxh, table, outh, rows, nchunks, base):
      def body(j, carry):
        off = base + j * CH
        pltpu.sync_copy(idxh.at[pl.ds(off, CH)], idx_v)
        pltpu.async_copy(table.at[idx_v], rows, sem).wait()
        pltpu.sync_copy(rows, outh.at[pl.ds(off, CH)])
        return carry
      lax.fori_loop(0, nchunks, body, 0)

    qbase = wid * (B // NW)
    bbase = wid * (T * B // NW)
    gather_pass(u0h, t_u0, ue0o, rows32, nq, qbase)
    gather_pass(u1h, t_u1, ue1o, rows32, nq, qbase)
    gather_pass(u2h, t_u2, ue2o, rows32, nq, qbase)
    gather_pass(qih, t_it, qio, rows64, nq, qbase)
    gather_pass(qch, t_ct, qco, rows64, nq, qbase)
    gather_pass(bih, t_it, bio, rows64, nb, bbase)
    gather_pass(bch, t_ct, bco, rows64, nb, bbase)

  return k(u0, u1, u2, qit, qct, bit, bct,
           emb_user_id, emb_user_city, emb_user_age, emb_item, emb_cate)


def _prelu(x, a):
  return jnp.where(x >= 0.0, x, a * x)


def _dot(x, w):
  return jnp.dot(x, w, preferred_element_type=jnp.float32)


def _tc_body(Bb,
             ud_r, isf_r, ue0_r, ue1_r, ue2_r, qi_r, qc_r,
             bi_r, bc_r, mk_r,
             WAi_r, WAc_r, WBi_r, WBc_r, WDi_r, WDc_r,
             ab0_r, aa0_r, aW1_r, ab1_r, aa1_r, aWf_r, abf_r,
             Fud_r, Fisf_r, Fue0_r, Fue1_r, Fue2_r, Fqi_r, Fqc_r,
             Fai_r, Fac_r, fb0_r, fa0_r, fW1_r, fb1_r, fa1_r,
             oW_r, ob_r, out_r):
  qi = qi_r[...]
  qc = qc_r[...]
  bi3 = bi_r[...]
  bc3 = bc_r[...]

  bir = bi3.reshape(T * Bb, 64)
  bcr = bc3.reshape(T * Bb, 64)
  pir = (bi3 * qi[None, :, :]).reshape(T * Bb, 64)
  pcr = (bc3 * qc[None, :, :]).reshape(T * Bb, 64)

  hq = _dot(qi, WAi_r[...]) + _dot(qc, WAc_r[...])
  h0 = (jnp.broadcast_to(hq[None], (T, Bb, 80)).reshape(T * Bb, 80)
        + _dot(bir, WBi_r[...]) + _dot(bcr, WBc_r[...])
        + _dot(pir, WDi_r[...]) + _dot(pcr, WDc_r[...]) + ab0_r[...])
  h0 = _prelu(h0, aa0_r[...])
  h1 = _prelu(_dot(h0, aW1_r[...]) + ab1_r[...], aa1_r[...])
  s = _dot(h1, aWf_r[...]) + abf_r[...]
  s3 = s.reshape(T, Bb, 1)
  s3 = jnp.where(mk_r[...] == 0.0, jnp.float32(-4294967295.0), s3)
  m = jnp.max(s3, axis=0, keepdims=True)
  e = jnp.exp(s3 - m)
  w3 = e / jnp.sum(e, axis=0, keepdims=True)
  atti = jnp.sum(w3 * bi3, axis=0)
  attc = jnp.sum(w3 * bc3, axis=0)

  h2 = (_dot(ud_r[...], Fud_r[...]) + _dot(isf_r[...], Fisf_r[...])
        + _dot(ue0_r[...], Fue0_r[...]) + _dot(ue1_r[...], Fue1_r[...])
        + _dot(ue2_r[...], Fue2_r[...])
        + _dot(qi, Fqi_r[...]) + _dot(qc, Fqc_r[...])
        + _dot(atti, Fai_r[...]) + _dot(attc, Fac_r[...]) + fb0_r[...])
  h2 = _prelu(h2, fa0_r[...])
  h3 = _prelu(_dot(h2, fW1_r[...]) + fb1_r[...], fa1_r[...])
  out_r[...] = jax.nn.sigmoid(_dot(h3, oW_r[...]) + ob_r[...])


def _tc_dense(ud, isf, ue0, ue1, ue2, qi, qc, bi3, bc3, mk3, weights,
              interpret=False):
  B = ud.shape[0]
  Bb = 512 if B % 512 == 0 else B
  grid = (B // Bb,)

  def rows(n):
    return pl.BlockSpec((Bb, n), lambda i: (i, 0))

  def full(a):
    return pl.BlockSpec(a.shape, lambda i: (0,) * a.ndim)

  in_specs = [
      rows(5), rows(3), rows(32), rows(32), rows(32), rows(64), rows(64),
      pl.BlockSpec((T, Bb, 64), lambda i: (0, i, 0)),
      pl.BlockSpec((T, Bb, 64), lambda i: (0, i, 0)),
      pl.BlockSpec((T, Bb, 1), lambda i: (0, i, 0)),
  ] + [full(w) for w in weights]

  return pl.pallas_call(
      functools.partial(_tc_body, Bb),
      grid=grid,
      in_specs=in_specs,
      out_specs=pl.BlockSpec((Bb, 1), lambda i: (i, 0)),
      out_shape=jax.ShapeDtypeStruct((B, 1), jnp.float32),
      interpret=interpret,
  )(ud, isf, ue0, ue1, ue2, qi, qc, bi3, bc3, mk3, *weights)


def _prep_weights(att_W0, att_b0, att_a0, att_W1, att_b1, att_a1,
                  att_Wf, att_bf, bn_gamma, bn_beta,
                  ffn_W0, ffn_b0, ffn_a0, ffn_W1, ffn_b1, ffn_a1,
                  out_W, out_b):
  A, Bm, C, D = (att_W0[0:128], att_W0[128:256],
                 att_W0[256:384], att_W0[384:512])
  AC = A + C
  BC = Bm - C
  g = bn_gamma / math.sqrt(1.0 + 1e-3)
  F = ffn_W0 * g[:, None]
  fb0 = ffn_b0 + bn_beta @ ffn_W0
  r = lambda v: v.reshape(1, -1)
  return [
      AC[0:64], AC[64:128], BC[0:64], BC[64:128], D[0:64], D[64:128],
      r(att_b0), r(att_a0), att_W1, r(att_b1), r(att_a1), att_Wf, r(att_bf),
      F[0:5], F[101:104], F[5:37], F[37:69], F[69:101],
      F[104:168], F[168:232], F[232:296], F[296:360],
      r(fb0), r(ffn_a0), ffn_W1, r(ffn_b1), r(ffn_a1), out_W, r(out_b),
  ]


def kernel(user_dense_input, user_sparse_input, item_dense_input,
           item_sparse_input, behavior_input, emb_user_id, emb_user_city,
           emb_user_age, emb_item, emb_cate, att_W0, att_b0, att_a0,
           att_W1, att_b1, att_a1, att_Wf, att_bf, bn_gamma, bn_beta,
           ffn_W0, ffn_b0, ffn_a0, ffn_W1, ffn_b1, ffn_a1, out_W, out_b):
  B = user_dense_input.shape[0]
  us = user_sparse_input.astype(jnp.int32)
  its = item_sparse_input.astype(jnp.int32)
  beh = behavior_input.astype(jnp.int32).reshape(B, T, 3)

  u0, u1, u2 = us[:, 0], us[:, 1], us[:, 2]
  qit, qct = its[:, 0], its[:, 1]
  bit = beh[:, :, 1].T.reshape(T * B)
  bct = beh[:, :, 2].T.reshape(T * B)
  mk3 = beh[:, :, 0].T.astype(jnp.float32).reshape(T, B, 1)

  ue0, ue1, ue2, qi, qc, bi, bc = _sc_gather_all(
      u0, u1, u2, qit, qct, bit, bct,
      emb_user_id, emb_user_city, emb_user_age, emb_item, emb_cate)

  weights = _prep_weights(att_W0, att_b0, att_a0, att_W1, att_b1, att_a1,
                          att_Wf, att_bf, bn_gamma, bn_beta,
                          ffn_W0, ffn_b0, ffn_a0, ffn_W1, ffn_b1, ffn_a1,
                          out_W, out_b)

  return _tc_dense(user_dense_input, its.astype(jnp.float32),
                   ue0, ue1, ue2, qi, qc,
                   bi.reshape(T, B, 64), bc.reshape(T, B, 64), mk3, weights)
